# baseline (speedup 1.0000x reference)
from functools import partial

import jax
import jax.numpy as jnp
from jax.experimental import pallas as pl


def _mm_kernel(n_batches, x_ref, l_ref, o_ref):
    l = l_ref[...]
    for n in range(n_batches):
        o_ref[n] = jnp.dot(l, x_ref[n], preferred_element_type=jnp.float32)


def kernel(x, L):
    N, M, Fin = x.shape
    Mp = L.shape[0]
    TM = 512
    xb = x.astype(jnp.bfloat16)
    lb = L.astype(jnp.bfloat16)
    out = pl.pallas_call(
        partial(_mm_kernel, N),
        grid=(Mp // TM,),
        in_specs=[
            pl.BlockSpec((N, M, Fin), lambda i: (0, 0, 0)),
            pl.BlockSpec((TM, M), lambda i: (i, 0)),
        ],
        out_specs=pl.BlockSpec((N, TM, Fin), lambda i: (0, i, 0)),
        out_shape=jax.ShapeDtypeStruct((N, Mp, Fin), jnp.float32),
    )(xb, lb)
    return out


# speedup vs baseline: 1.3120x; 1.3120x over previous
from functools import partial

import jax
import jax.numpy as jnp
from jax.experimental import pallas as pl


def _mm_kernel(n_batches, x_ref, l_ref, o_ref):
    l = l_ref[...].astype(jnp.bfloat16)
    for n in range(n_batches):
        o_ref[n] = jnp.dot(l, x_ref[n], preferred_element_type=jnp.float32)


def kernel(x, L):
    N, M, Fin = x.shape
    Mp = L.shape[0]
    TM = 512
    xb = x.astype(jnp.bfloat16)
    out = pl.pallas_call(
        partial(_mm_kernel, N),
        grid=(Mp // TM,),
        in_specs=[
            pl.BlockSpec((N, M, Fin), lambda i: (0, 0, 0)),
            pl.BlockSpec((TM, M), lambda i: (i, 0)),
        ],
        out_specs=pl.BlockSpec((N, TM, Fin), lambda i: (0, i, 0)),
        out_shape=jax.ShapeDtypeStruct((N, Mp, Fin), jnp.float32),
    )(xb, L)
    return out


# speedup vs baseline: 1.4710x; 1.1212x over previous
from functools import partial

import jax
import jax.numpy as jnp
from jax.experimental import pallas as pl


def _mm_kernel(n_batches, x_ref, l_ref, o_ref):
    l = l_ref[...].astype(jnp.bfloat16)
    for n in range(n_batches):
        o_ref[n] = jnp.dot(
            l, x_ref[n].astype(jnp.bfloat16), preferred_element_type=jnp.float32
        )


def kernel(x, L):
    N, M, Fin = x.shape
    Mp = L.shape[0]
    TM = 256
    out = pl.pallas_call(
        partial(_mm_kernel, N),
        grid=(Mp // TM,),
        in_specs=[
            pl.BlockSpec((N, M, Fin), lambda i: (0, 0, 0)),
            pl.BlockSpec((TM, M), lambda i: (i, 0)),
        ],
        out_specs=pl.BlockSpec((N, TM, Fin), lambda i: (0, i, 0)),
        out_shape=jax.ShapeDtypeStruct((N, Mp, Fin), jnp.float32),
    )(x, L)
    return out


# speedup vs baseline: 1.4927x; 1.0147x over previous
from functools import partial

import jax
import jax.numpy as jnp
from jax.experimental import pallas as pl
from jax.experimental.pallas import tpu as pltpu


def _mm_kernel(n_batches, nk, tk, x_hbm, l_ref, o_ref, xb_ref, stage_ref, sem):
    i = pl.program_id(0)
    l = l_ref[...].astype(jnp.bfloat16)

    @pl.when(i == 0)
    def _first_step():
        for k in range(min(2, nk)):
            pltpu.make_async_copy(
                x_hbm.at[:, pl.ds(k * tk, tk), :], stage_ref.at[k], sem.at[k]
            ).start()
        for k in range(nk):
            slot = k % 2
            pltpu.make_async_copy(
                x_hbm.at[:, pl.ds(k * tk, tk), :], stage_ref.at[slot], sem.at[slot]
            ).wait()
            xb_ref[:, pl.ds(k * tk, tk), :] = stage_ref[slot].astype(jnp.bfloat16)
            if k + 2 < nk:
                pltpu.make_async_copy(
                    x_hbm.at[:, pl.ds((k + 2) * tk, tk), :],
                    stage_ref.at[slot],
                    sem.at[slot],
                ).start()
            lk = l[:, k * tk : (k + 1) * tk]
            for n in range(n_batches):
                d = jnp.dot(
                    lk,
                    xb_ref[n, pl.ds(k * tk, tk), :],
                    preferred_element_type=jnp.float32,
                )
                if k == 0:
                    o_ref[n] = d
                else:
                    o_ref[n] += d

    @pl.when(i > 0)
    def _rest():
        for n in range(n_batches):
            o_ref[n] = jnp.dot(l, xb_ref[n], preferred_element_type=jnp.float32)


def kernel(x, L):
    N, M, Fin = x.shape
    Mp = L.shape[0]
    TM = 512
    TK = 512
    out = pl.pallas_call(
        partial(_mm_kernel, N, M // TK, TK),
        grid=(Mp // TM,),
        in_specs=[
            pl.BlockSpec(memory_space=pl.MemorySpace.ANY),
            pl.BlockSpec((TM, M), lambda i: (i, 0)),
        ],
        out_specs=pl.BlockSpec((N, TM, Fin), lambda i: (0, i, 0)),
        out_shape=jax.ShapeDtypeStruct((N, Mp, Fin), jnp.float32),
        scratch_shapes=[
            pltpu.VMEM((N, M, Fin), jnp.bfloat16),
            pltpu.VMEM((2, N, TK, Fin), jnp.float32),
            pltpu.SemaphoreType.DMA((2,)),
        ],
    )(x, L)
    return out
